# baseline (speedup 1.0000x reference)
import functools

import jax
import jax.numpy as jnp
from jax import lax
from jax.experimental import pallas as pl
from jax.experimental.pallas import tpu as pltpu
from jax.experimental.pallas import tpu_sc as plsc

_HEADS = 16
_N = 32
_R = 2 * _N - 1
_L = 16
_NC = 2
_Q = _HEADS * _N * _N // 4
_W0 = _HEADS * 64


def _bias_body(tab_hbm, out_hbm, tab_v, out_v, sem1, sem2):
    sid = lax.axis_index("s")

    pltpu.sync_copy(tab_hbm, tab_v)

    def quarter(q, wid, base):
        for h in range(q * 4, q * 4 + 4):
            for c in range(2):
                rv = tab_v[pl.ds(_W0 + h * 64 + 16 * c + (_N - 1) - wid, _L)]
                for i in range(_N):
                    bh = tab_v[pl.ds(h * 64 + 16 * c + (_N - 1) - i, _L)]
                    out_v[pl.ds(base + (h * _N + i) * _N + 16 * c, _L)] = bh + rv

    copies = []
    sems = [sem1, sem2, sem1, sem2]
    for half in range(2):
        wid = sid * 2 + half
        for q in range(4):
            quarter(q, wid, half * 4 * _Q)
            copies.append(pltpu.async_copy(
                out_v.at[pl.ds((half * 4 + q) * _Q, _Q)],
                out_hbm.at[wid, pl.ds(q * _Q, _Q)],
                sems[half * 2 + q // 2]))
    for cp in copies:
        cp.wait()


_bias_kernel = functools.partial(
    pl.kernel,
    mesh=plsc.VectorSubcoreMesh(core_axis_name="c", subcore_axis_name="s", num_cores=1),
    out_type=jax.ShapeDtypeStruct((_N, _HEADS * _N * _N), jnp.float32),
    scratch_types=[
        pltpu.VMEM((2 * _HEADS * 64,), jnp.float32),
        pltpu.VMEM((2 * _HEADS * _N * _N,), jnp.float32),
        pltpu.SemaphoreType.DMA,
        pltpu.SemaphoreType.DMA,
    ],
)(_bias_body)


def kernel(rel_height, rel_width, H, W):
    del H, W
    pad = jnp.zeros((_HEADS, 1), jnp.float32)
    rht = jnp.concatenate([rel_height.T, pad], axis=1)
    rwt = jnp.concatenate([rel_width[::-1].T, pad], axis=1)
    tab = jnp.concatenate([rht, rwt], axis=0).reshape(-1)
    out = _bias_kernel(tab)
    return out.reshape(_N, _HEADS, _N, _N)


# speedup vs baseline: 1.0517x; 1.0517x over previous
import functools

import jax
import jax.numpy as jnp
from jax import lax
from jax.experimental import pallas as pl
from jax.experimental.pallas import tpu as pltpu
from jax.experimental.pallas import tpu_sc as plsc

_HEADS = 16
_N = 32
_R = 2 * _N - 1
_L = 16
_NC = 2
_Q = _HEADS * _N * _N // 4
_W0 = _HEADS * 64


def _bias_body(tab_hbm, out_hbm, tab_v, out_v, sem1, sem2):
    wid = lax.axis_index("s") * _NC + lax.axis_index("c")

    pltpu.sync_copy(tab_hbm, tab_v)

    def quarter(q):
        for h in range(q * 4, q * 4 + 4):
            for c in range(2):
                rv = tab_v[pl.ds(_W0 + h * 64 + 16 * c + (_N - 1) - wid, _L)]
                for i in range(_N):
                    bh = tab_v[pl.ds(h * 64 + 16 * c + (_N - 1) - i, _L)]
                    out_v[pl.ds((h * _N + i) * _N + 16 * c, _L)] = bh + rv

    copies = []
    sems = [sem1, sem2, sem1, sem2]
    for q in range(4):
        quarter(q)
        copies.append(pltpu.async_copy(
            out_v.at[pl.ds(q * _Q, _Q)], out_hbm.at[wid, pl.ds(q * _Q, _Q)],
            sems[q]))
    for cp in copies:
        cp.wait()


_bias_kernel = functools.partial(
    pl.kernel,
    mesh=plsc.VectorSubcoreMesh(core_axis_name="c", subcore_axis_name="s"),
    out_type=jax.ShapeDtypeStruct((_N, _HEADS * _N * _N), jnp.float32),
    scratch_types=[
        pltpu.VMEM((2 * _HEADS * 64,), jnp.float32),
        pltpu.VMEM((_HEADS * _N * _N,), jnp.float32),
        pltpu.SemaphoreType.DMA,
        pltpu.SemaphoreType.DMA,
    ],
)(_bias_body)


def kernel(rel_height, rel_width, H, W):
    del H, W
    pad = jnp.zeros((_HEADS, 1), jnp.float32)
    rht = jnp.concatenate([rel_height.T, pad], axis=1)
    rwt = jnp.concatenate([rel_width[::-1].T, pad], axis=1)
    tab = jnp.concatenate([rht, rwt], axis=0).reshape(-1)
    out = _bias_kernel(tab)
    return out.reshape(_N, _HEADS, _N, _N)


# speedup vs baseline: 1.2296x; 1.1691x over previous
import functools

import jax
import jax.numpy as jnp
from jax import lax
from jax.experimental import pallas as pl
from jax.experimental.pallas import tpu as pltpu
from jax.experimental.pallas import tpu_sc as plsc

_HEADS = 16
_N = 32
_R = 2 * _N - 1
_L = 16
_NC = 2
_Q = _HEADS * _N * _N // 4
_W0 = _HEADS * 64


def _bias_body(tab_hbm, out_hbm, tab_v, out_v, sem1, sem2):
    wid = lax.axis_index("s") * _NC + lax.axis_index("c")

    pltpu.sync_copy(tab_hbm, tab_v)

    def quarter(q):
        for h in range(q * 4, q * 4 + 4):
            for c in range(2):
                rv = tab_v[pl.ds(_W0 + h * 64 + 16 * c + (_N - 1) - wid, _L)]
                for i in range(_N):
                    bh = tab_v[pl.ds(h * 64 + 16 * c + (_N - 1) - i, _L)]
                    out_v[h, i, pl.ds(16 * c, _L)] = bh + rv

    copies = []
    sems = [sem1, sem2, sem1, sem2]
    for q in range(4):
        quarter(q)
        copies.append(pltpu.async_copy(
            out_v.at[pl.ds(q * 4, 4)],
            out_hbm.at[wid, pl.ds(q * 4, 4)],
            sems[q]))
    for cp in copies:
        cp.wait()


_bias_kernel = functools.partial(
    pl.kernel,
    mesh=plsc.VectorSubcoreMesh(core_axis_name="c", subcore_axis_name="s"),
    out_type=jax.ShapeDtypeStruct((_N, _HEADS, _N, _N), jnp.float32),
    scratch_types=[
        pltpu.VMEM((2 * _HEADS * 64,), jnp.float32),
        pltpu.VMEM((_HEADS, _N, _N), jnp.float32),
        pltpu.SemaphoreType.DMA,
        pltpu.SemaphoreType.DMA,
    ],
)(_bias_body)


def kernel(rel_height, rel_width, H, W):
    del H, W
    pad = jnp.zeros((_HEADS, 1), jnp.float32)
    rht = jnp.concatenate([rel_height.T, pad], axis=1)
    rwt = jnp.concatenate([rel_width[::-1].T, pad], axis=1)
    tab = jnp.concatenate([rht, rwt], axis=0).reshape(-1)
    return _bias_kernel(tab)
